# baseline (speedup 1.0000x reference)
import jax
import jax.numpy as jnp
from jax.experimental import pallas as pl
from jax.experimental.pallas import tpu as pltpu

_CHUNKS = 8


def _dma_body(pe_hbm, out_hbm, *sems):
    rows = out_hbm.shape[0]
    per = rows // _CHUNKS
    copies = []
    for c in range(_CHUNKS):
        cp = pltpu.make_async_copy(
            pe_hbm.at[pl.ds(c * per, per)],
            out_hbm.at[pl.ds(c * per, per)],
            sems[c],
        )
        cp.start()
        copies.append(cp)
    for cp in copies:
        cp.wait()


def kernel(x, pe):
    seq_len = x.shape[1]
    d = pe.shape[2]
    pe2 = pe.reshape(pe.shape[1], d)
    out = pl.pallas_call(
        _dma_body,
        in_specs=[pl.BlockSpec(memory_space=pltpu.MemorySpace.HBM)],
        out_specs=pl.BlockSpec(memory_space=pltpu.MemorySpace.HBM),
        out_shape=jax.ShapeDtypeStruct((seq_len, d), pe.dtype),
        scratch_shapes=[pltpu.SemaphoreType.DMA] * _CHUNKS,
    )(pe2)
    return out.reshape(1, seq_len, d)


# speedup vs baseline: 34.9573x; 34.9573x over previous
import jax
import jax.numpy as jnp
from jax.experimental import pallas as pl


def _copy_body(pe_ref, out_ref):
    out_ref[...] = pe_ref[...]


def kernel(x, pe):
    seq_len = x.shape[1]
    d = pe.shape[2]
    pe2 = pe.reshape(pe.shape[1], d)
    block = 512
    out = pl.pallas_call(
        _copy_body,
        grid=(seq_len // block,),
        in_specs=[pl.BlockSpec((block, d), lambda i: (i, 0))],
        out_specs=pl.BlockSpec((block, d), lambda i: (i, 0)),
        out_shape=jax.ShapeDtypeStruct((seq_len, d), pe.dtype),
    )(pe2)
    return out.reshape(1, seq_len, d)


# speedup vs baseline: 42.6091x; 1.2189x over previous
import jax
import jax.numpy as jnp
from jax.experimental import pallas as pl


def _copy_body(pe_ref, out_ref):
    out_ref[...] = pe_ref[...]


def kernel(x, pe):
    seq_len = x.shape[1]
    d = pe.shape[2]
    pe2 = pe.reshape(pe.shape[1], d)
    block = 1024
    out = pl.pallas_call(
        _copy_body,
        grid=(seq_len // block,),
        in_specs=[pl.BlockSpec((block, d), lambda i: (i, 0))],
        out_specs=pl.BlockSpec((block, d), lambda i: (i, 0)),
        out_shape=jax.ShapeDtypeStruct((seq_len, d), pe.dtype),
    )(pe2)
    return out.reshape(1, seq_len, d)
